# baseline (speedup 1.0000x reference)
import functools

import jax
import jax.numpy as jnp
from jax import lax
from jax.experimental import pallas as pl
from jax.experimental.pallas import tpu as pltpu
from jax.experimental.pallas import tpu_sc as plsc


def kernel(x):
    B, N, S, D = x.shape
    info = plsc.get_sparse_core_info()
    NC, NS = info.num_cores, info.num_subcores
    NW = NC * NS
    pairs = B * 2
    w_per_pair = NW // pairs
    chunk = S // w_per_pair
    mesh = plsc.VectorSubcoreMesh(core_axis_name="c", subcore_axis_name="s")

    nsub = 4
    sub = chunk // nsub

    @functools.partial(
        pl.kernel,
        mesh=mesh,
        out_type=jax.ShapeDtypeStruct((B, 2, S, D), x.dtype),
        scratch_types=(
            [pltpu.VMEM((sub, D), x.dtype) for _ in range(nsub)]
            + [pltpu.SemaphoreType.DMA for _ in range(2 * nsub)]
        ),
    )
    def k(x_hbm, out_hbm, *scratch):
        bufs = scratch[:nsub]
        in_sems = scratch[nsub : 2 * nsub]
        out_sems = scratch[2 * nsub :]
        wid = lax.axis_index("s") * NC + lax.axis_index("c")
        pair = wid // w_per_pair
        slot = wid % w_per_pair
        b = pair // 2
        j = pair % 2
        off = slot * chunk
        gathers = [
            pltpu.async_copy(
                x_hbm.at[b, 2 * j, pl.ds(off + i * sub, sub)], bufs[i], in_sems[i]
            )
            for i in range(nsub)
        ]
        scatters = []
        for i in range(nsub):
            gathers[i].wait()
            scatters.append(
                pltpu.async_copy(
                    bufs[i], out_hbm.at[b, j, pl.ds(off + i * sub, sub)], out_sems[i]
                )
            )
        for s in scatters:
            s.wait()

    return k(x)


# speedup vs baseline: 1.0009x; 1.0009x over previous
import functools

import jax
import jax.numpy as jnp
from jax import lax
from jax.experimental import pallas as pl
from jax.experimental.pallas import tpu as pltpu
from jax.experimental.pallas import tpu_sc as plsc


def kernel(x):
    B, N, S, D = x.shape
    info = plsc.get_sparse_core_info()
    NC, NS = info.num_cores, info.num_subcores
    NW = NC * NS
    pairs = B * 2
    w_per_pair = NW // pairs
    chunk = S // w_per_pair
    mesh = plsc.VectorSubcoreMesh(core_axis_name="c", subcore_axis_name="s")

    nsub = 4
    sub = chunk // nsub

    @functools.partial(
        pl.kernel,
        mesh=mesh,
        out_type=jax.ShapeDtypeStruct((B, 2, S, D), x.dtype),
        compiler_params=pltpu.CompilerParams(use_tc_tiling_on_sc=True),
        scratch_types=(
            [pltpu.VMEM((sub, D), x.dtype) for _ in range(nsub)]
            + [pltpu.SemaphoreType.DMA for _ in range(2 * nsub)]
        ),
    )
    def k(x_hbm, out_hbm, *scratch):
        bufs = scratch[:nsub]
        in_sems = scratch[nsub : 2 * nsub]
        out_sems = scratch[2 * nsub :]
        wid = lax.axis_index("s") * NC + lax.axis_index("c")
        pair = wid // w_per_pair
        slot = wid % w_per_pair
        b = pair // 2
        j = pair % 2
        off = slot * chunk
        gathers = [
            pltpu.async_copy(
                x_hbm.at[b, 2 * j, pl.ds(off + i * sub, sub)], bufs[i], in_sems[i]
            )
            for i in range(nsub)
        ]
        scatters = []
        for i in range(nsub):
            gathers[i].wait()
            scatters.append(
                pltpu.async_copy(
                    bufs[i], out_hbm.at[b, j, pl.ds(off + i * sub, sub)], out_sems[i]
                )
            )
        for s in scatters:
            s.wait()

    return k(x)


# speedup vs baseline: 1.9673x; 1.9656x over previous
import jax
import jax.numpy as jnp
from jax.experimental import pallas as pl
from jax.experimental.pallas import tpu as pltpu

_SB = 1024


def _copy_body(x_ref, o_ref):
    o_ref[...] = x_ref[...]


def kernel(x):
    B, N, S, D = x.shape
    return pl.pallas_call(
        _copy_body,
        grid=(B, 2, S // _SB),
        in_specs=[pl.BlockSpec((1, 1, _SB, D), lambda b, j, s: (b, 2 * j, s, 0))],
        out_specs=pl.BlockSpec((1, 1, _SB, D), lambda b, j, s: (b, j, s, 0)),
        out_shape=jax.ShapeDtypeStruct((B, 2, S, D), x.dtype),
        compiler_params=pltpu.CompilerParams(
            dimension_semantics=("parallel", "parallel", "parallel")
        ),
    )(x)


# speedup vs baseline: 3.4419x; 1.7496x over previous
import jax
import jax.numpy as jnp
from jax.experimental import pallas as pl
from jax.experimental.pallas import tpu as pltpu

_SB = 4096


def _copy_body(x_ref, o_ref):
    o_ref[...] = x_ref[...]


def kernel(x):
    B, N, S, D = x.shape
    return pl.pallas_call(
        _copy_body,
        grid=(B, 2, S // _SB),
        in_specs=[pl.BlockSpec((1, 1, _SB, D), lambda b, j, s: (b, 2 * j, s, 0))],
        out_specs=pl.BlockSpec((1, 1, _SB, D), lambda b, j, s: (b, j, s, 0)),
        out_shape=jax.ShapeDtypeStruct((B, 2, S, D), x.dtype),
        compiler_params=pltpu.CompilerParams(
            dimension_semantics=("parallel", "parallel", "parallel")
        ),
    )(x)


# speedup vs baseline: 4.0873x; 1.1875x over previous
import jax
import jax.numpy as jnp
from jax.experimental import pallas as pl
from jax.experimental.pallas import tpu as pltpu

_NSPLIT = 2


def _dma_body(x_ref, o_ref, *scratch):
    B = x_ref.shape[0]
    S = x_ref.shape[2]
    c = S // _NSPLIT
    n = B * 2 * _NSPLIT
    bufs = scratch[:n]
    in_sems = scratch[n : 2 * n]
    out_sems = scratch[2 * n :]
    srcs, dsts = [], []
    for b in range(B):
        for j in range(2):
            for i in range(_NSPLIT):
                srcs.append(x_ref.at[b, 2 * j, pl.ds(i * c, c)])
                dsts.append(o_ref.at[b, j, pl.ds(i * c, c)])
    gathers = [
        pltpu.make_async_copy(srcs[k], bufs[k], in_sems[k]) for k in range(n)
    ]
    for g in gathers:
        g.start()
    scatters = []
    for k in range(n):
        gathers[k].wait()
        s = pltpu.make_async_copy(bufs[k], dsts[k], out_sems[k])
        s.start()
        scatters.append(s)
    for s in scatters:
        s.wait()


def kernel(x):
    B, N, S, D = x.shape
    c = S // _NSPLIT
    n = B * 2 * _NSPLIT
    return pl.pallas_call(
        _dma_body,
        in_specs=[pl.BlockSpec(memory_space=pl.ANY)],
        out_specs=pl.BlockSpec(memory_space=pl.ANY),
        out_shape=jax.ShapeDtypeStruct((B, 2, S, D), x.dtype),
        scratch_shapes=(
            [pltpu.VMEM((c, D), x.dtype) for _ in range(n)]
            + [pltpu.SemaphoreType.DMA for _ in range(2 * n)]
        ),
    )(x)
